# Initial kernel scaffold; baseline (speedup 1.0000x reference)
import jax
import jax.numpy as jnp
from jax.experimental import pallas as pl


def kernel(node_feat, coord_feat, edge_feat, edge_index, W1, b1, W2, b2, eps):
    raise NotImplementedError("write your pallas kernel here")


# speedup vs baseline: 3.3681x; 3.3681x over previous
import functools

import jax
import jax.numpy as jnp
from jax import lax
from jax.experimental import pallas as pl
from jax.experimental.pallas import tpu as pltpu
from jax.experimental.pallas import tpu_sc as plsc

N = 10000
E = 160000
D = 256
Q = 64
CH = 128
NC = 2
NS = 16
E_PAD = 163840
NCHUNK = E_PAD // CH
TCH = NCHUNK // NS
R = 8
GRP = TCH // R
NP = 10240
PT = NP // NS

_mesh = plsc.VectorSubcoreMesh(core_axis_name="c", subcore_axis_name="s")


@functools.partial(
    pl.kernel,
    out_type=[
        jax.ShapeDtypeStruct((4, NP, Q), jnp.float32),
        jax.ShapeDtypeStruct((NP, 16), jnp.float32),
    ],
    mesh=_mesh,
    scratch_types=[
        pltpu.VMEM((R, CH), jnp.int32),
        pltpu.VMEM((R, CH), jnp.int32),
        pltpu.VMEM((R, CH, Q), jnp.float32),
        pltpu.VMEM((CH, 16), jnp.float32),
        pltpu.VMEM_SHARED((NP, Q), jnp.float32),
        pltpu.VMEM_SHARED((NP, 16), jnp.float32),
        pltpu.SemaphoreType.DMA,
        pltpu.SemaphoreType.DMA,
        pltpu.SemaphoreType.DMA,
    ],
    compiler_params=pltpu.CompilerParams(use_tc_tiling_on_sc=False),
)
def _sc_aggregate(nfq_hbm, srcq_hbm, dst_hbm, out_sum, out_deg,
                  isrc, idst, rows, ones_b, acc_sh, deg_sh, gsem, ssem, dsem):
    c = lax.axis_index("c")
    s = lax.axis_index("s")
    zeros16 = jnp.zeros((16,), jnp.float32)
    ones16 = jnp.ones((16,), jnp.float32)
    base_r = s * PT

    for r in range(2):
        q = 2 * r + c

        def zrow(i, _):
            for j in range(Q // 16):
                rows[0, i, pl.ds(j * 16, 16)] = zeros16
            return 0
        lax.fori_loop(0, CH, zrow, 0)
        for t in range(PT // CH):
            pltpu.sync_copy(rows.at[0], acc_sh.at[pl.ds(base_r + t * CH, CH)])
        if r == 0:
            def zdeg(i, _):
                ones_b[i] = zeros16
                return 0
            lax.fori_loop(0, CH, zdeg, 0)

            @pl.when(c == 0)
            def _():
                for t in range(PT // CH):
                    pltpu.sync_copy(ones_b,
                                    deg_sh.at[pl.ds(base_r + t * CH, CH)])

            def orow(i, _):
                ones_b[i] = ones16
                return 0
            lax.fori_loop(0, CH, orow, 0)
        plsc.subcore_barrier()

        def group(g, _):
            cb = s * TCH + g * R
            pltpu.sync_copy(srcq_hbm.at[q, pl.ds(cb, R)], isrc)
            pltpu.sync_copy(dst_hbm.at[pl.ds(cb, R)], idst)
            gd = [pltpu.async_copy(nfq_hbm.at[isrc.at[k]], rows.at[k], gsem)
                  for k in range(R)]
            sd = []
            dd = []
            for k in range(R):
                gd[k].wait()
                sd.append(pltpu.async_copy(rows.at[k], acc_sh.at[idst.at[k]],
                                           ssem, add=True))
                if r == 0:
                    dd.append((k, idst.at[k]))
            if r == 0:
                @pl.when(c == 0)
                def _():
                    dds = [pltpu.async_copy(ones_b, deg_sh.at[ix], dsem,
                                            add=True) for _k, ix in dd]
                    for d in dds:
                        d.wait()
            for d in sd:
                d.wait()
            return 0
        lax.fori_loop(0, GRP, group, 0)
        plsc.subcore_barrier()

        for t in range(PT // CH):
            r0 = base_r + t * CH
            pltpu.sync_copy(acc_sh.at[pl.ds(r0, CH)], rows.at[0])
            pltpu.sync_copy(rows.at[0], out_sum.at[q, pl.ds(r0, CH)])

    @pl.when(c == 0)
    def _():
        for t in range(PT // CH):
            r0 = base_r + t * CH
            pltpu.sync_copy(deg_sh.at[pl.ds(r0, CH)], ones_b)
            pltpu.sync_copy(ones_b, out_deg.at[pl.ds(r0, CH)])


_BLK = 1000


def _tc_body(nf, sm, dg, w1, b1, w2, b2, eps, out):
    deg = jnp.maximum(dg[...], 1.0)
    mean = sm[...] / deg
    rst = (1.0 + eps[0, 0]) * nf[...] + mean
    h = jnp.maximum(
        jnp.dot(rst, w1[...], preferred_element_type=jnp.float32) + b1[...], 0.0)
    out[...] = jnp.dot(h, w2[...], preferred_element_type=jnp.float32) + b2[...]


def _tc_apply(nf, sm, deg, W1, b1, W2, b2, eps):
    return pl.pallas_call(
        _tc_body,
        grid=(N // _BLK,),
        in_specs=[
            pl.BlockSpec((_BLK, D), lambda i: (i, 0)),
            pl.BlockSpec((_BLK, D), lambda i: (i, 0)),
            pl.BlockSpec((_BLK, 1), lambda i: (i, 0)),
            pl.BlockSpec((D, D), lambda i: (0, 0)),
            pl.BlockSpec((1, D), lambda i: (0, 0)),
            pl.BlockSpec((D, D), lambda i: (0, 0)),
            pl.BlockSpec((1, D), lambda i: (0, 0)),
            pl.BlockSpec((1, 1), lambda i: (0, 0)),
        ],
        out_specs=pl.BlockSpec((_BLK, D), lambda i: (i, 0)),
        out_shape=jax.ShapeDtypeStruct((N, D), jnp.float32),
    )(nf, sm, deg, W1, b1, W2, b2, eps)


def kernel(node_feat, coord_feat, edge_feat, edge_index, W1, b1, W2, b2, eps):
    src = edge_index[0].astype(jnp.int32)
    dst = edge_index[1].astype(jnp.int32)
    pad = E_PAD - E
    src_p = jnp.concatenate([src, jnp.zeros((pad,), jnp.int32)])
    dst_p = jnp.concatenate([dst, jnp.full((pad,), N, jnp.int32)])
    srcq = (src_p[None, :]
            + (jnp.arange(4, dtype=jnp.int32) * N)[:, None]).reshape(4, NCHUNK, CH)
    dst2 = dst_p.reshape(NCHUNK, CH)
    nfq = node_feat.reshape(N, 4, Q).transpose(1, 0, 2).reshape(4 * N, Q)

    out_sum, out_deg = _sc_aggregate(nfq, srcq, dst2)
    summed = jnp.concatenate([out_sum[i, :N] for i in range(4)], axis=1)
    deg = out_deg[:N, 0:1]

    hx = _tc_apply(node_feat, summed, deg, W1, jnp.reshape(b1, (1, D)),
                   W2, jnp.reshape(b2, (1, D)),
                   jnp.reshape(eps, (1, 1)).astype(jnp.float32))
    return (hx, coord_feat, edge_feat)


# speedup vs baseline: 3.4844x; 1.0345x over previous
import functools

import jax
import jax.numpy as jnp
from jax import lax
from jax.experimental import pallas as pl
from jax.experimental.pallas import tpu as pltpu
from jax.experimental.pallas import tpu_sc as plsc

N = 10000
E = 160000
D = 256
Q = 64
CH = 128
NC = 2
NS = 16
E_PAD = 163840
NCHUNK = E_PAD // CH
TCH = NCHUNK // NS
R = 8
GRP = TCH // R
NP = 10240
PT = NP // NS

_mesh = plsc.VectorSubcoreMesh(core_axis_name="c", subcore_axis_name="s")


def _make_round(with_deg):
    out_type = [jax.ShapeDtypeStruct((NC, NP, Q), jnp.float32)]
    scratch = [
        pltpu.VMEM((R, CH), jnp.int32),
        pltpu.VMEM((R, CH), jnp.int32),
        pltpu.VMEM((R, CH, Q), jnp.float32),
        pltpu.VMEM((CH, 16), jnp.float32),
        pltpu.VMEM_SHARED((NP, Q), jnp.float32),
        pltpu.SemaphoreType.DMA,
        pltpu.SemaphoreType.DMA,
        pltpu.SemaphoreType.DMA,
    ]
    if with_deg:
        out_type.append(jax.ShapeDtypeStruct((NC, NP, 16), jnp.float32))
        scratch.insert(5, pltpu.VMEM_SHARED((NP, 16), jnp.float32))

    @functools.partial(
        pl.kernel,
        out_type=out_type,
        mesh=_mesh,
        scratch_types=scratch,
        compiler_params=pltpu.CompilerParams(use_tc_tiling_on_sc=False),
    )
    def _round(nfq_hbm, src2_hbm, dst_hbm, out_sum, *rest):
        if with_deg:
            (out_deg, isrc, idst, rows, ones_b, acc_sh, deg_sh,
             gsem, ssem, dsem) = rest
        else:
            isrc, idst, rows, ones_b, acc_sh, gsem, ssem, dsem = rest
        c = lax.axis_index("c")
        s = lax.axis_index("s")
        zeros16 = jnp.zeros((16,), jnp.float32)
        ones16 = jnp.ones((16,), jnp.float32)
        base_r = s * PT

        def zrow(i, _):
            for j in range(Q // 16):
                rows[0, i, pl.ds(j * 16, 16)] = zeros16
            return 0
        lax.fori_loop(0, CH, zrow, 0)
        for t in range(PT // CH):
            pltpu.sync_copy(rows.at[0], acc_sh.at[pl.ds(base_r + t * CH, CH)])
        if with_deg:
            def zdeg(i, _):
                ones_b[i] = zeros16
                return 0
            lax.fori_loop(0, CH, zdeg, 0)
            for t in range(PT // CH):
                pltpu.sync_copy(ones_b, deg_sh.at[pl.ds(base_r + t * CH, CH)])

            def orow(i, _):
                ones_b[i] = ones16
                return 0
            lax.fori_loop(0, CH, orow, 0)
        plsc.subcore_barrier()

        def group(g, _):
            cb = s * TCH + g * R
            pltpu.sync_copy(src2_hbm.at[c, pl.ds(cb, R)], isrc)
            pltpu.sync_copy(dst_hbm.at[pl.ds(cb, R)], idst)
            gd = [pltpu.async_copy(nfq_hbm.at[isrc.at[k]], rows.at[k], gsem)
                  for k in range(R)]
            sd = []
            for k in range(R):
                gd[k].wait()
                sd.append(pltpu.async_copy(rows.at[k],
                                           acc_sh.at[idst.at[k]],
                                           ssem, add=True))
            if with_deg:
                @pl.when(lax.rem(g, 2) == c)
                def _():
                    dds = [pltpu.async_copy(ones_b, deg_sh.at[idst.at[cb + k]],
                                            dsem, add=True) for k in range(R)]
                    for d in dds:
                        d.wait()
            for d in sd:
                d.wait()
            return 0
        lax.fori_loop(0, GRP, group, 0)
        plsc.subcore_barrier()

        for t in range(PT // CH):
            r0 = base_r + t * CH
            pltpu.sync_copy(acc_sh.at[pl.ds(r0, CH)], rows.at[0])
            pltpu.sync_copy(rows.at[0], out_sum.at[c, pl.ds(r0, CH)])
        if with_deg:
            for t in range(PT // CH):
                r0 = base_r + t * CH
                pltpu.sync_copy(deg_sh.at[pl.ds(r0, CH)], ones_b)
                pltpu.sync_copy(ones_b, out_deg.at[c, pl.ds(r0, CH)])

    return _round


TCHD = NCHUNK // (NC * NS)
RD = 8
GRPD = TCHD // RD


@functools.partial(
    pl.kernel,
    out_type=jax.ShapeDtypeStruct((NC, NP, 16), jnp.float32),
    mesh=_mesh,
    scratch_types=[
        pltpu.VMEM((TCHD, CH), jnp.int32),
        pltpu.VMEM((CH, 16), jnp.float32),
        pltpu.VMEM_SHARED((NP, 16), jnp.float32),
        pltpu.SemaphoreType.DMA,
    ],
    compiler_params=pltpu.CompilerParams(use_tc_tiling_on_sc=False),
)
def _sc_degree(dst_hbm, out_deg, idst, ones_b, deg_sh, dsem):
    c = lax.axis_index("c")
    s = lax.axis_index("s")
    zeros16 = jnp.zeros((16,), jnp.float32)
    ones16 = jnp.ones((16,), jnp.float32)
    base_r = s * PT

    pltpu.sync_copy(dst_hbm.at[pl.ds((c * NS + s) * TCHD, TCHD)], idst)

    def zdeg(i, _):
        ones_b[i] = zeros16
        return 0
    lax.fori_loop(0, CH, zdeg, 0)
    for t in range(PT // CH):
        pltpu.sync_copy(ones_b, deg_sh.at[pl.ds(base_r + t * CH, CH)])

    def orow(i, _):
        ones_b[i] = ones16
        return 0
    lax.fori_loop(0, CH, orow, 0)
    plsc.subcore_barrier()

    def group(g, _):
        cb = g * RD
        dds = [pltpu.async_copy(ones_b, deg_sh.at[idst.at[cb + k]],
                                dsem, add=True) for k in range(RD)]
        for d in dds:
            d.wait()
        return 0
    lax.fori_loop(0, GRPD, group, 0)
    plsc.subcore_barrier()

    for t in range(PT // CH):
        r0 = base_r + t * CH
        pltpu.sync_copy(deg_sh.at[pl.ds(r0, CH)], ones_b)
        pltpu.sync_copy(ones_b, out_deg.at[c, pl.ds(r0, CH)])


_round0 = _make_round(False)
_round1 = _make_round(False)

_BLK = 1000


def _tc_body(nf, sm, dg, w1, b1, w2, b2, eps, out):
    deg = jnp.maximum(dg[...], 1.0)
    mean = sm[...] / deg
    rst = (1.0 + eps[0, 0]) * nf[...] + mean
    h = jnp.maximum(
        jnp.dot(rst, w1[...], preferred_element_type=jnp.float32) + b1[...], 0.0)
    out[...] = jnp.dot(h, w2[...], preferred_element_type=jnp.float32) + b2[...]


def _tc_apply(nf, sm, deg, W1, b1, W2, b2, eps):
    return pl.pallas_call(
        _tc_body,
        grid=(N // _BLK,),
        in_specs=[
            pl.BlockSpec((_BLK, D), lambda i: (i, 0)),
            pl.BlockSpec((_BLK, D), lambda i: (i, 0)),
            pl.BlockSpec((_BLK, 1), lambda i: (i, 0)),
            pl.BlockSpec((D, D), lambda i: (0, 0)),
            pl.BlockSpec((1, D), lambda i: (0, 0)),
            pl.BlockSpec((D, D), lambda i: (0, 0)),
            pl.BlockSpec((1, D), lambda i: (0, 0)),
            pl.BlockSpec((1, 1), lambda i: (0, 0)),
        ],
        out_specs=pl.BlockSpec((_BLK, D), lambda i: (i, 0)),
        out_shape=jax.ShapeDtypeStruct((N, D), jnp.float32),
    )(nf, sm, deg, W1, b1, W2, b2, eps)


def kernel(node_feat, coord_feat, edge_feat, edge_index, W1, b1, W2, b2, eps):
    src = edge_index[0].astype(jnp.int32)
    dst = edge_index[1].astype(jnp.int32)
    pad = E_PAD - E
    src_p = jnp.concatenate([src, jnp.zeros((pad,), jnp.int32)])
    dst_p = jnp.concatenate([dst, jnp.full((pad,), N, jnp.int32)])
    srcq = (src_p[None, :]
            + (jnp.arange(4, dtype=jnp.int32) * N)[:, None]).reshape(4, NCHUNK, CH)
    dst2 = dst_p.reshape(NCHUNK, CH)
    nfq = node_feat.reshape(N, 4, Q).transpose(1, 0, 2).reshape(4 * N, Q)

    (sum01,) = _round0(nfq, srcq[0:2], dst2)
    (sum23,) = _round1(nfq, srcq[2:4], dst2)
    deg2 = _sc_degree(dst2)
    summed = jnp.concatenate(
        [sum01[0, :N], sum01[1, :N], sum23[0, :N], sum23[1, :N]], axis=1)
    deg = (deg2[0, :N, 0] + deg2[1, :N, 0])[:, None]

    hx = _tc_apply(node_feat, summed, deg, W1, jnp.reshape(b1, (1, D)),
                   W2, jnp.reshape(b2, (1, D)),
                   jnp.reshape(eps, (1, 1)).astype(jnp.float32))
    return (hx, coord_feat, edge_feat)


# speedup vs baseline: 3.5752x; 1.0261x over previous
import functools

import jax
import jax.numpy as jnp
from jax import lax
from jax.experimental import pallas as pl
from jax.experimental.pallas import tpu as pltpu
from jax.experimental.pallas import tpu_sc as plsc

N = 10000
E = 160000
D = 256
Q = 64
CH = 128
NC = 2
NS = 16
E_PAD = 163840
NCHUNK = E_PAD // CH
TCH = NCHUNK // NS
R = 8
GRP = TCH // R
NP = 10240
PT = NP // NS

_mesh = plsc.VectorSubcoreMesh(core_axis_name="c", subcore_axis_name="s")


def _make_round(with_deg):
    out_type = [jax.ShapeDtypeStruct((NC, NP, Q), jnp.float32)]
    scratch = [
        pltpu.VMEM((2, R, CH), jnp.int32),
        pltpu.VMEM((2, R, CH), jnp.int32),
        pltpu.VMEM((R, CH, Q), jnp.float32),
        pltpu.VMEM((CH, 16), jnp.float32),
        pltpu.VMEM_SHARED((NP, Q), jnp.float32),
        pltpu.SemaphoreType.DMA,
        pltpu.SemaphoreType.DMA,
        pltpu.SemaphoreType.DMA,
        pltpu.SemaphoreType.DMA,
    ]
    if with_deg:
        out_type.append(jax.ShapeDtypeStruct((NC, NP, 16), jnp.float32))
        scratch.insert(5, pltpu.VMEM_SHARED((NP, 16), jnp.float32))

    @functools.partial(
        pl.kernel,
        out_type=out_type,
        mesh=_mesh,
        scratch_types=scratch,
        compiler_params=pltpu.CompilerParams(use_tc_tiling_on_sc=False),
    )
    def _round(nfq_hbm, src2_hbm, dst_hbm, out_sum, *rest):
        if with_deg:
            (out_deg, isrc, idst, rows, ones_b, acc_sh, deg_sh,
             gsem, ssem, dsem, isem) = rest
        else:
            isrc, idst, rows, ones_b, acc_sh, gsem, ssem, dsem, isem = rest
        c = lax.axis_index("c")
        s = lax.axis_index("s")
        zeros16 = jnp.zeros((16,), jnp.float32)
        ones16 = jnp.ones((16,), jnp.float32)
        base_r = s * PT

        def zrow(i, _):
            for j in range(Q // 16):
                rows[0, i, pl.ds(j * 16, 16)] = zeros16
            return 0
        lax.fori_loop(0, CH, zrow, 0)
        for t in range(PT // CH):
            pltpu.sync_copy(rows.at[0], acc_sh.at[pl.ds(base_r + t * CH, CH)])
        if with_deg:
            def zdeg(i, _):
                ones_b[i] = zeros16
                return 0
            lax.fori_loop(0, CH, zdeg, 0)
            for t in range(PT // CH):
                pltpu.sync_copy(ones_b, deg_sh.at[pl.ds(base_r + t * CH, CH)])

            def orow(i, _):
                ones_b[i] = ones16
                return 0
            lax.fori_loop(0, CH, orow, 0)
        plsc.subcore_barrier()

        def run_group(b):
            gd = [pltpu.async_copy(nfq_hbm.at[isrc.at[b, k]], rows.at[k], gsem)
                  for k in range(R)]
            sd = []
            for k in range(R):
                gd[k].wait()
                sd.append(pltpu.async_copy(rows.at[k],
                                           acc_sh.at[idst.at[b, k]],
                                           ssem, add=True))
            for d in sd:
                d.wait()

        def fetch_idx(g, b):
            cb = s * TCH + g * R
            ia = pltpu.async_copy(src2_hbm.at[c, pl.ds(cb, R)],
                                  isrc.at[b], isem)
            ib = pltpu.async_copy(dst_hbm.at[pl.ds(cb, R)],
                                  idst.at[b], isem)
            return ia, ib

        ia0, ib0 = fetch_idx(0, 0)
        ia0.wait()
        ib0.wait()

        def pair(p, _):
            ga = 2 * p
            da, db = fetch_idx(ga + 1, 1)
            run_group(0)
            da.wait()
            db.wait()

            @pl.when(p < GRP // 2 - 1)
            def _():
                fetch_idx(ga + 2, 0)
            run_group(1)

            @pl.when(p < GRP // 2 - 1)
            def _():
                pltpu.make_async_copy(src2_hbm.at[c, pl.ds(0, R)],
                                      isrc.at[0], isem).wait()
                pltpu.make_async_copy(dst_hbm.at[pl.ds(0, R)],
                                      idst.at[0], isem).wait()
            return 0
        lax.fori_loop(0, GRP // 2, pair, 0)
        plsc.subcore_barrier()

        for t in range(PT // CH):
            r0 = base_r + t * CH
            pltpu.sync_copy(acc_sh.at[pl.ds(r0, CH)], rows.at[0])
            pltpu.sync_copy(rows.at[0], out_sum.at[c, pl.ds(r0, CH)])
        if with_deg:
            for t in range(PT // CH):
                r0 = base_r + t * CH
                pltpu.sync_copy(deg_sh.at[pl.ds(r0, CH)], ones_b)
                pltpu.sync_copy(ones_b, out_deg.at[c, pl.ds(r0, CH)])

    return _round


TCHD = NCHUNK // (NC * NS)
RD = 8
GRPD = TCHD // RD


@functools.partial(
    pl.kernel,
    out_type=jax.ShapeDtypeStruct((NC, NP, 16), jnp.float32),
    mesh=_mesh,
    scratch_types=[
        pltpu.VMEM((TCHD, CH), jnp.int32),
        pltpu.VMEM((CH, 16), jnp.float32),
        pltpu.VMEM_SHARED((NP, 16), jnp.float32),
        pltpu.SemaphoreType.DMA,
    ],
    compiler_params=pltpu.CompilerParams(use_tc_tiling_on_sc=False),
)
def _sc_degree(dst_hbm, out_deg, idst, ones_b, deg_sh, dsem):
    c = lax.axis_index("c")
    s = lax.axis_index("s")
    zeros16 = jnp.zeros((16,), jnp.float32)
    ones16 = jnp.ones((16,), jnp.float32)
    base_r = s * PT

    pltpu.sync_copy(dst_hbm.at[pl.ds((c * NS + s) * TCHD, TCHD)], idst)

    def zdeg(i, _):
        ones_b[i] = zeros16
        return 0
    lax.fori_loop(0, CH, zdeg, 0)
    for t in range(PT // CH):
        pltpu.sync_copy(ones_b, deg_sh.at[pl.ds(base_r + t * CH, CH)])

    def orow(i, _):
        ones_b[i] = ones16
        return 0
    lax.fori_loop(0, CH, orow, 0)
    plsc.subcore_barrier()

    def group(g, _):
        cb = g * RD
        dds = [pltpu.async_copy(ones_b, deg_sh.at[idst.at[cb + k]],
                                dsem, add=True) for k in range(RD)]
        for d in dds:
            d.wait()
        return 0
    lax.fori_loop(0, GRPD, group, 0)
    plsc.subcore_barrier()

    for t in range(PT // CH):
        r0 = base_r + t * CH
        pltpu.sync_copy(deg_sh.at[pl.ds(r0, CH)], ones_b)
        pltpu.sync_copy(ones_b, out_deg.at[c, pl.ds(r0, CH)])


_round0 = _make_round(False)
_round1 = _make_round(False)

_BLK = 1000


def _tc_body(nf, sm, dg, w1, b1, w2, b2, eps, out):
    deg = jnp.maximum(dg[...], 1.0)
    mean = sm[...] / deg
    rst = (1.0 + eps[0, 0]) * nf[...] + mean
    h = jnp.maximum(
        jnp.dot(rst, w1[...], preferred_element_type=jnp.float32) + b1[...], 0.0)
    out[...] = jnp.dot(h, w2[...], preferred_element_type=jnp.float32) + b2[...]


def _tc_apply(nf, sm, deg, W1, b1, W2, b2, eps):
    return pl.pallas_call(
        _tc_body,
        grid=(N // _BLK,),
        in_specs=[
            pl.BlockSpec((_BLK, D), lambda i: (i, 0)),
            pl.BlockSpec((_BLK, D), lambda i: (i, 0)),
            pl.BlockSpec((_BLK, 1), lambda i: (i, 0)),
            pl.BlockSpec((D, D), lambda i: (0, 0)),
            pl.BlockSpec((1, D), lambda i: (0, 0)),
            pl.BlockSpec((D, D), lambda i: (0, 0)),
            pl.BlockSpec((1, D), lambda i: (0, 0)),
            pl.BlockSpec((1, 1), lambda i: (0, 0)),
        ],
        out_specs=pl.BlockSpec((_BLK, D), lambda i: (i, 0)),
        out_shape=jax.ShapeDtypeStruct((N, D), jnp.float32),
    )(nf, sm, deg, W1, b1, W2, b2, eps)


def kernel(node_feat, coord_feat, edge_feat, edge_index, W1, b1, W2, b2, eps):
    src = edge_index[0].astype(jnp.int32)
    dst = edge_index[1].astype(jnp.int32)
    pad = E_PAD - E
    src_p = jnp.concatenate([src, jnp.zeros((pad,), jnp.int32)])
    dst_p = jnp.concatenate([dst, jnp.full((pad,), N, jnp.int32)])
    srcq = (src_p[None, :]
            + (jnp.arange(4, dtype=jnp.int32) * N)[:, None]).reshape(4, NCHUNK, CH)
    dst2 = dst_p.reshape(NCHUNK, CH)
    nfq = node_feat.reshape(N, 4, Q).transpose(1, 0, 2).reshape(4 * N, Q)

    (sum01,) = _round0(nfq, srcq[0:2], dst2)
    (sum23,) = _round1(nfq, srcq[2:4], dst2)
    deg2 = _sc_degree(dst2)
    summed = jnp.concatenate(
        [sum01[0, :N], sum01[1, :N], sum23[0, :N], sum23[1, :N]], axis=1)
    deg = (deg2[0, :N, 0] + deg2[1, :N, 0])[:, None]

    hx = _tc_apply(node_feat, summed, deg, W1, jnp.reshape(b1, (1, D)),
                   W2, jnp.reshape(b2, (1, D)),
                   jnp.reshape(eps, (1, 1)).astype(jnp.float32))
    return (hx, coord_feat, edge_feat)
